# baseline (speedup 1.0000x reference)
import jax
import jax.numpy as jnp
from jax.experimental import pallas as pl
from jax.experimental.pallas import tpu as pltpu

_B, _C, _H, _W = 128, 3, 224, 224
_ROWS = (_H * _W) // 128


def _gray_body(inds_ref, x_ref, o_ref):
    b = pl.program_id(0)
    sel = inds_ref[b] != 0

    @pl.when(sel)
    def _():
        L = (x_ref[0, 0] * (299.0 / 1000.0)
             + x_ref[0, 1] * (587.0 / 1000.0)
             + x_ref[0, 2] * (114.0 / 1000.0))
        o_ref[0, 0] = L
        o_ref[0, 1] = L
        o_ref[0, 2] = L

    @pl.when(jnp.logical_not(sel))
    def _():
        o_ref[...] = x_ref[...]


def kernel(x, inds):
    xr = x.reshape(_B, _C, _ROWS, 128)
    out = pl.pallas_call(
        _gray_body,
        grid_spec=pltpu.PrefetchScalarGridSpec(
            num_scalar_prefetch=1,
            grid=(_B,),
            in_specs=[pl.BlockSpec((1, _C, _ROWS, 128), lambda b, inds: (b, 0, 0, 0))],
            out_specs=pl.BlockSpec((1, _C, _ROWS, 128), lambda b, inds: (b, 0, 0, 0)),
        ),
        out_shape=jax.ShapeDtypeStruct((_B, _C, _ROWS, 128), jnp.float32),
    )(inds.astype(jnp.int32), xr)
    return out.reshape(_B, _C, _H, _W)
